# baseline (speedup 1.0000x reference)
import jax
import jax.numpy as jnp
from jax import lax
from jax.experimental import pallas as pl
from jax.experimental.pallas import tpu as pltpu
from jax.experimental.pallas import tpu_sc as plsc

NC, NS, L = 2, 16, 16
NW = NC * NS

B, WL, D = 4096, 20, 128
BW = B // NW
PAIRS = BW * WL
CHUNK = 128
NCHUNK = PAIRS // CHUNK


def _body(tgt_hbm, win_hbm, nadj_hbm, emb_hbm, out_hbm,
          t_stage, w_stage, nadj_v, node_v,
          tgt_idx, win_idx, neg_idx,
          tb0, pb0, nb0, tb1, pb1, nb1, acc_v,
          sem0, sem1, semh):
    wid = lax.axis_index("s") * NC + lax.axis_index("c")

    pltpu.sync_copy(tgt_hbm.at[pl.ds(wid * BW, BW)], t_stage)
    pltpu.sync_copy(win_hbm.at[pl.ds(wid * BW, BW)], w_stage)

    def _nodes(i, carry):
        j = lax.iota(jnp.int32, L) + i * L
        node_v[pl.ds(i * L, L)] = plsc.load_gather(
            t_stage, [j, (j & 0) + (WL - 1)])
        return carry
    lax.fori_loop(0, BW // L, _nodes, jnp.int32(0))

    nadj_cps = []
    for bb in range(BW):
        vec = node_v[pl.ds((bb // L) * L, L)]
        n_s = vec[bb % L]
        nadj_cps.append(pltpu.async_copy(
            nadj_hbm.at[pl.ds(n_s, 1)], nadj_v.at[pl.ds(bb, 1)], semh))

    wl_vec = jnp.full((L,), WL, jnp.int32)
    def _permute_tw(k, carry):
        p = lax.iota(jnp.int32, L) + k * L
        b = lax.div(p, wl_vec)
        l = p - b * wl_vec
        sl = pl.ds((k & 7) * L, L)
        tgt_idx[k >> 3, sl] = plsc.load_gather(t_stage, [b, l])
        win_idx[k >> 3, sl] = plsc.load_gather(w_stage, [b, l])
        return carry
    lax.fori_loop(0, PAIRS // L, _permute_tw, jnp.int32(0))

    slots = ((tb0, pb0, nb0, sem0), (tb1, pb1, nb1, sem1))

    def start_tw(c, slot):
        tb, pb, _, sem = slot
        return (
            pltpu.async_copy(emb_hbm.at[tgt_idx.at[c]], tb, sem),
            pltpu.async_copy(emb_hbm.at[win_idx.at[c]], pb, sem),
        )

    def start_neg(c, slot):
        _, _, nb, sem = slot
        return (pltpu.async_copy(emb_hbm.at[neg_idx.at[c]], nb, sem),)

    pending = {0: start_tw(0, slots[0]), 1: start_tw(1, slots[1])}

    for cp in nadj_cps:
        cp.wait()

    def _permute_neg(k, carry):
        p = lax.iota(jnp.int32, L) + k * L
        b = lax.div(p, wl_vec)
        l = p - b * wl_vec
        neg_idx[k >> 3, pl.ds((k & 7) * L, L)] = \
            plsc.load_gather(nadj_v, [b, l])
        return carry
    lax.fori_loop(0, PAIRS // L, _permute_neg, jnp.int32(0))

    pending[0] += start_neg(0, slots[0])
    pending[1] += start_neg(1, slots[1])

    def compute(slot, acc):
        tb, pb, nb, _ = slot
        def row(r, a):
            t = []
            for q in range(D // L):
                sl = pl.ds(q * L, L)
                t.append(tb[r, sl] * (nb[r, sl] - pb[r, sl]))
            while len(t) > 1:
                t = [t[i] + t[i + 1] for i in range(0, len(t) - 1, 2)] \
                    + ([t[-1]] if len(t) % 2 else [])
            return a + t[0]
        return plsc.parallel_loop(0, CHUNK, unroll=2, carry=acc)(row)

    acc = jnp.zeros((L,), jnp.float32)
    for c in range(NCHUNK):
        for cp in pending.pop(c):
            cp.wait()
        acc = compute(slots[c % 2], acc)
        if c + 2 < NCHUNK:
            pending[c + 2] = (start_tw(c + 2, slots[c % 2])
                              + start_neg(c + 2, slots[c % 2]))

    acc_v[...] = acc
    pltpu.sync_copy(acc_v, out_hbm.at[wid])


def kernel(target, window, non_adj_list, embed_table):
    mesh = plsc.VectorSubcoreMesh(
        core_axis_name="c", subcore_axis_name="s",
        num_cores=NC, num_subcores=NS)
    partials = pl.kernel(
        _body,
        out_type=jax.ShapeDtypeStruct((NW, L), jnp.float32),
        mesh=mesh,
        compiler_params=pltpu.CompilerParams(
            needs_layout_passes=False, use_tc_tiling_on_sc=False),
        scratch_types=[
            pltpu.VMEM((BW, WL), jnp.int32),
            pltpu.VMEM((BW, WL), jnp.int32),
            pltpu.VMEM((BW, WL), jnp.int32),
            pltpu.VMEM((BW,), jnp.int32),
            pltpu.VMEM((NCHUNK, CHUNK), jnp.int32),
            pltpu.VMEM((NCHUNK, CHUNK), jnp.int32),
            pltpu.VMEM((NCHUNK, CHUNK), jnp.int32),
            pltpu.VMEM((CHUNK, D), jnp.float32),
            pltpu.VMEM((CHUNK, D), jnp.float32),
            pltpu.VMEM((CHUNK, D), jnp.float32),
            pltpu.VMEM((CHUNK, D), jnp.float32),
            pltpu.VMEM((CHUNK, D), jnp.float32),
            pltpu.VMEM((CHUNK, D), jnp.float32),
            pltpu.VMEM((L,), jnp.float32),
            pltpu.SemaphoreType.DMA,
            pltpu.SemaphoreType.DMA,
            pltpu.SemaphoreType.DMA,
        ],
    )(target, window, non_adj_list, embed_table)
    return jnp.sum(partials)


# speedup vs baseline: 1.4085x; 1.4085x over previous
import jax
import jax.numpy as jnp
from jax import lax
from jax.experimental import pallas as pl
from jax.experimental.pallas import tpu as pltpu
from jax.experimental.pallas import tpu_sc as plsc

NC, NS, L = 2, 16, 16
NW = NC * NS

B, WL, D = 4096, 20, 128
BW = B // NW
PAIRS = BW * WL
CHUNK = 64
NCHUNK = PAIRS // CHUNK


def _body(tgt_hbm, win_hbm, nadj_hbm, emb_hbm, out_hbm,
          t_stage, w_stage, nadj_v, node_v,
          tgt_idx, win_idx, neg_idx,
          tb0, pb0, nb0, tb1, pb1, nb1, acc_v,
          sem0, sem1, semh):
    wid = lax.axis_index("s") * NC + lax.axis_index("c")

    pltpu.sync_copy(tgt_hbm.at[pl.ds(wid * BW, BW)], t_stage)
    pltpu.sync_copy(win_hbm.at[pl.ds(wid * BW, BW)], w_stage)

    def _nodes(i, carry):
        j = lax.iota(jnp.int32, L) + i * L
        node_v[pl.ds(i * L, L)] = plsc.load_gather(
            t_stage, [j, (j & 0) + (WL - 1)])
        return carry
    lax.fori_loop(0, BW // L, _nodes, jnp.int32(0))

    nadj_cps = []
    for bb in range(BW):
        vec = node_v[pl.ds((bb // L) * L, L)]
        n_s = vec[bb % L]
        nadj_cps.append(pltpu.async_copy(
            nadj_hbm.at[pl.ds(n_s, 1)], nadj_v.at[pl.ds(bb, 1)], semh))

    wl_vec = jnp.full((L,), WL, jnp.int32)
    def _permute_tw(k, carry):
        p = lax.iota(jnp.int32, L) + k * L
        b = lax.div(p, wl_vec)
        l = p - b * wl_vec
        sl = pl.ds((k & 3) * L, L)
        tgt_idx[k >> 2, sl] = plsc.load_gather(t_stage, [b, l])
        win_idx[k >> 2, sl] = plsc.load_gather(w_stage, [b, l])
        return carry
    lax.fori_loop(0, PAIRS // L, _permute_tw, jnp.int32(0))

    slots = ((tb0, pb0, nb0, sem0), (tb1, pb1, nb1, sem1))

    def start_tw(c, slot):
        tb, pb, _, sem = slot
        return (
            pltpu.async_copy(emb_hbm.at[tgt_idx.at[c]], tb, sem),
            pltpu.async_copy(emb_hbm.at[win_idx.at[c]], pb, sem),
        )

    def start_neg(c, slot):
        _, _, nb, sem = slot
        return (pltpu.async_copy(emb_hbm.at[neg_idx.at[c]], nb, sem),)

    pending = {0: start_tw(0, slots[0]), 1: start_tw(1, slots[1])}

    for cp in nadj_cps:
        cp.wait()

    def _permute_neg(k, carry):
        p = lax.iota(jnp.int32, L) + k * L
        b = lax.div(p, wl_vec)
        l = p - b * wl_vec
        neg_idx[k >> 2, pl.ds((k & 3) * L, L)] = \
            plsc.load_gather(nadj_v, [b, l])
        return carry
    lax.fori_loop(0, PAIRS // L, _permute_neg, jnp.int32(0))

    pending[0] += start_neg(0, slots[0])
    pending[1] += start_neg(1, slots[1])

    def compute(slot, acc):
        tb, pb, nb, _ = slot
        def row(r, a):
            t = []
            for q in range(D // L):
                sl = pl.ds(q * L, L)
                t.append(tb[r, sl] * (nb[r, sl] - pb[r, sl]))
            while len(t) > 1:
                t = [t[i] + t[i + 1] for i in range(0, len(t) - 1, 2)] \
                    + ([t[-1]] if len(t) % 2 else [])
            return a + t[0]
        return plsc.parallel_loop(0, CHUNK, unroll=2, carry=acc)(row)

    acc = jnp.zeros((L,), jnp.float32)
    for c in range(NCHUNK):
        for cp in pending.pop(c):
            cp.wait()
        acc = compute(slots[c % 2], acc)
        if c + 2 < NCHUNK:
            pending[c + 2] = (start_tw(c + 2, slots[c % 2])
                              + start_neg(c + 2, slots[c % 2]))

    acc_v[...] = acc
    pltpu.sync_copy(acc_v, out_hbm.at[wid])


def kernel(target, window, non_adj_list, embed_table):
    mesh = plsc.VectorSubcoreMesh(
        core_axis_name="c", subcore_axis_name="s",
        num_cores=NC, num_subcores=NS)
    partials = pl.kernel(
        _body,
        out_type=jax.ShapeDtypeStruct((NW, L), jnp.float32),
        mesh=mesh,
        compiler_params=pltpu.CompilerParams(
            needs_layout_passes=False, use_tc_tiling_on_sc=True),
        scratch_types=[
            pltpu.VMEM((BW, WL), jnp.int32),
            pltpu.VMEM((BW, WL), jnp.int32),
            pltpu.VMEM((BW, WL), jnp.int32),
            pltpu.VMEM((BW,), jnp.int32),
            pltpu.VMEM((NCHUNK, CHUNK), jnp.int32),
            pltpu.VMEM((NCHUNK, CHUNK), jnp.int32),
            pltpu.VMEM((NCHUNK, CHUNK), jnp.int32),
            pltpu.VMEM((CHUNK, D), jnp.float32),
            pltpu.VMEM((CHUNK, D), jnp.float32),
            pltpu.VMEM((CHUNK, D), jnp.float32),
            pltpu.VMEM((CHUNK, D), jnp.float32),
            pltpu.VMEM((CHUNK, D), jnp.float32),
            pltpu.VMEM((CHUNK, D), jnp.float32),
            pltpu.VMEM((L,), jnp.float32),
            pltpu.SemaphoreType.DMA,
            pltpu.SemaphoreType.DMA,
            pltpu.SemaphoreType.DMA,
        ],
    )(target, window, non_adj_list, embed_table)
    return jnp.sum(partials)
